# baseline (speedup 1.0000x reference)
import functools

import jax
import jax.numpy as jnp
from jax.experimental import pallas as pl
from jax.experimental.pallas import tpu as pltpu


def _dense_moe_body(x_ref, gw_ref, w1_ref, b1_ref, w2_ref, b2_ref,
                    out_ref, comb_ref, acc_ref, *, n_e, n_ff):
    e = pl.program_id(1)
    f = pl.program_id(2)

    @pl.when((e == 0) & (f == 0))
    def _gate():
        xb = x_ref[...]
        logits = jax.lax.dot_general(
            xb, gw_ref[...], (((1,), (1,)), ((), ())),
            preferred_element_type=jnp.float32)
        m = jnp.max(logits, axis=-1, keepdims=True)
        p = jnp.exp(logits - m)
        w = p / jnp.sum(p, axis=-1, keepdims=True)
        E = w.shape[-1]
        iota = jax.lax.broadcasted_iota(jnp.int32, w.shape, 1)
        w0 = jnp.max(w, axis=-1, keepdims=True)
        i0 = jnp.min(jnp.where(w == w0, iota, E), axis=-1, keepdims=True)
        mask0 = iota == i0
        w_excl = jnp.where(mask0, -jnp.inf, w)
        w1v = jnp.max(w_excl, axis=-1, keepdims=True)
        i1 = jnp.min(jnp.where(w_excl == w1v, iota, E), axis=-1, keepdims=True)
        mask1 = iota == i1
        denom = w0 + w1v
        comb_ref[...] = (jnp.where(mask0, w0, 0.0)
                         + jnp.where(mask1, w1v, 0.0)) / denom

    @pl.when((e == 0) & (f == 0))
    def _zero():
        acc_ref[...] = jnp.zeros_like(acc_ref)

    xb = x_ref[...]
    w1b = w1_ref[0]
    h = jax.lax.dot_general(xb, w1b, (((1,), (1,)), ((), ())),
                            preferred_element_type=jnp.float32)
    h = jnp.maximum(h + b1_ref[0], 0.0)
    w2b = w2_ref[0]
    part = jax.lax.dot_general(h, w2b, (((1,), (1,)), ((), ())),
                               preferred_element_type=jnp.float32)
    comb = comb_ref[...]
    eiota = jax.lax.broadcasted_iota(jnp.int32, comb.shape, 1)
    ce = jnp.sum(jnp.where(eiota == e, comb, 0.0), axis=-1, keepdims=True)
    acc_ref[...] += ce * part

    @pl.when(f == 0)
    def _bias2():
        acc_ref[...] += ce * b2_ref[0]

    @pl.when((e == n_e - 1) & (f == n_ff - 1))
    def _store():
        out_ref[...] = acc_ref[...]


def kernel(x, gate_W, W1, b1, W2, b2):
    B, T, C = x.shape
    E, F = W1.shape[0], W1.shape[1]
    N = B * T
    x_flat = x.reshape(N, C)

    bm = min(256, N)
    bf = min(512, F)
    n_t, n_e, n_ff = N // bm, E, F // bf
    b1r = b1.reshape(E * n_ff, 1, bf)
    b2r = b2.reshape(E, 1, C)

    out = pl.pallas_call(
        functools.partial(_dense_moe_body, n_e=n_e, n_ff=n_ff),
        grid=(n_t, n_e, n_ff),
        in_specs=[
            pl.BlockSpec((bm, C), lambda t, e, f: (t, 0)),
            pl.BlockSpec((E, C), lambda t, e, f: (0, 0)),
            pl.BlockSpec((1, bf, C), lambda t, e, f: (e, f, 0)),
            pl.BlockSpec((1, 1, bf), lambda t, e, f: (e * n_ff + f, 0, 0)),
            pl.BlockSpec((1, C, bf), lambda t, e, f: (e, 0, f)),
            pl.BlockSpec((1, 1, C), lambda t, e, f: (e, 0, 0)),
        ],
        out_specs=pl.BlockSpec((bm, C), lambda t, e, f: (t, 0)),
        out_shape=jax.ShapeDtypeStruct((N, C), jnp.float32),
        scratch_shapes=[
            pltpu.VMEM((bm, E), jnp.float32),
            pltpu.VMEM((bm, C), jnp.float32),
        ],
    )(x_flat, gate_W, W1, b1r, W2, b2r)
    return out.reshape(B, T, C)


# speedup vs baseline: 1.7933x; 1.7933x over previous
import functools

import jax
import jax.numpy as jnp
from jax.experimental import pallas as pl
from jax.experimental.pallas import tpu as pltpu

_BM = 256
_BF = 512


def _gating_body(x_ref, gw_ref, e0_ref, e1_ref, w0_ref, w1_ref):
    xb = x_ref[...]
    logits = jax.lax.dot_general(
        xb, gw_ref[...], (((1,), (1,)), ((), ())),
        preferred_element_type=jnp.float32)
    m = jnp.max(logits, axis=-1, keepdims=True)
    p = jnp.exp(logits - m)
    w = p / jnp.sum(p, axis=-1, keepdims=True)
    E = w.shape[-1]
    iota = jax.lax.broadcasted_iota(jnp.int32, w.shape, 1)
    w0 = jnp.max(w, axis=-1, keepdims=True)
    i0 = jnp.min(jnp.where(w == w0, iota, E), axis=-1, keepdims=True)
    w_excl = jnp.where(iota == i0, -jnp.inf, w)
    w1v = jnp.max(w_excl, axis=-1, keepdims=True)
    i1 = jnp.min(jnp.where(w_excl == w1v, iota, E), axis=-1, keepdims=True)
    denom = w0 + w1v
    e0_ref[...] = i0
    e1_ref[...] = i1
    w0_ref[...] = w0 / denom
    w1_ref[...] = w1v / denom


def _gating(x_flat, gate_W):
    N, C = x_flat.shape
    E = gate_W.shape[0]
    outs = pl.pallas_call(
        _gating_body,
        grid=(1,),
        in_specs=[
            pl.BlockSpec((N, C), lambda i: (0, 0)),
            pl.BlockSpec((E, C), lambda i: (0, 0)),
        ],
        out_specs=[
            pl.BlockSpec((N, 1), lambda i: (0, 0)),
            pl.BlockSpec((N, 1), lambda i: (0, 0)),
            pl.BlockSpec((N, 1), lambda i: (0, 0)),
            pl.BlockSpec((N, 1), lambda i: (0, 0)),
        ],
        out_shape=[
            jax.ShapeDtypeStruct((N, 1), jnp.int32),
            jax.ShapeDtypeStruct((N, 1), jnp.int32),
            jax.ShapeDtypeStruct((N, 1), jnp.float32),
            jax.ShapeDtypeStruct((N, 1), jnp.float32),
        ],
    )(x_flat, gate_W)
    return [o.reshape(N) for o in outs]


def _routing(e0, e1, w0, w1, E):
    N = e0.shape[0]
    e_all = jnp.concatenate([e0, e1])
    w_all = jnp.concatenate([w0, w1])
    order = jnp.argsort(e_all, stable=True)
    src_tokens = order % N
    scales = w_all[order]
    pos = jnp.argsort(order)
    group_sizes = jnp.sum(
        jax.nn.one_hot(e_all, E, dtype=jnp.int32), axis=0)
    return group_sizes, src_tokens, scales, pos


def _group_metadata(group_sizes, R, bm, E):
    g_max = R // bm + E - 1
    ends = jnp.cumsum(group_sizes)
    starts = ends - group_sizes
    ntiles = jnp.where(group_sizes > 0,
                       (ends - 1) // bm - starts // bm + 1, 0)
    total = jnp.sum(ntiles)
    tile_cum_excl = jnp.cumsum(ntiles) - ntiles
    posn = jnp.arange(g_max, dtype=jnp.int32)
    gid_raw = jnp.repeat(jnp.arange(E, dtype=jnp.int32), ntiles,
                         total_repeat_length=g_max)
    gid_last = jnp.take(gid_raw, total - 1)
    valid = posn < total
    gid = jnp.where(valid, gid_raw, gid_last)
    tid_raw = starts[gid] // bm + (posn - tile_cum_excl[gid])
    tid_last = jnp.take(tid_raw, total - 1)
    tid = jnp.where(valid, tid_raw, tid_last).astype(jnp.int32)
    goff = jnp.concatenate([jnp.zeros(1, jnp.int32),
                            ends.astype(jnp.int32)])
    return gid.astype(jnp.int32), tid, goff


def _gmm_body(gid_ref, tid_ref, goff_ref,
              xs_ref, w1_ref, b1_ref, w2_ref, b2_ref, sc_ref,
              y_ref, acc_ref, *, bm, n_ff):
    f = pl.program_id(0)
    g = pl.program_id(1)
    e = gid_ref[g]
    t = tid_ref[g]

    xb = xs_ref[...]
    w1b = w1_ref[0]
    h = jax.lax.dot_general(xb, w1b, (((1,), (1,)), ((), ())),
                            preferred_element_type=jnp.float32)
    h = jnp.maximum(h + b1_ref[0], 0.0)
    w2b = w2_ref[0]
    part = jax.lax.dot_general(h, w2b, (((1,), (1,)), ((), ())),
                               preferred_element_type=jnp.float32)

    rows = t * bm + jax.lax.broadcasted_iota(jnp.int32, (bm, 1), 0)
    mask = (rows >= goff_ref[e]) & (rows < goff_ref[e + 1])

    sl = pl.ds(t * bm, bm)
    prev = acc_ref[sl, :]
    acc = jnp.where(mask, jnp.where(f == 0, part, prev + part), prev)
    acc_ref[sl, :] = acc

    @pl.when(f == n_ff - 1)
    def _store():
        y_ref[...] = jnp.where(mask, (acc + b2_ref[0]) * sc_ref[...],
                               y_ref[...])


def _gmm(x_sorted, W1, b1, W2, b2, scales, gid, tid, goff):
    R, C = x_sorted.shape
    E, F = W1.shape[0], W1.shape[1]
    bm, bf = _BM, _BF
    n_ff = F // bf
    g_max = gid.shape[0]
    b1r = b1.reshape(E * n_ff, 1, bf)
    b2r = b2.reshape(E, 1, C)
    sc2 = scales.reshape(R, 1)

    grid_spec = pltpu.PrefetchScalarGridSpec(
        num_scalar_prefetch=3,
        grid=(n_ff, g_max),
        in_specs=[
            pl.BlockSpec((bm, C), lambda f, g, gid, tid, go: (tid[g], 0)),
            pl.BlockSpec((1, bf, C), lambda f, g, gid, tid, go: (gid[g], f, 0)),
            pl.BlockSpec((1, 1, bf),
                         lambda f, g, gid, tid, go: (gid[g] * n_ff + f, 0, 0)),
            pl.BlockSpec((1, C, bf), lambda f, g, gid, tid, go: (gid[g], 0, f)),
            pl.BlockSpec((1, 1, C), lambda f, g, gid, tid, go: (gid[g], 0, 0)),
            pl.BlockSpec((bm, 1), lambda f, g, gid, tid, go: (tid[g], 0)),
        ],
        out_specs=pl.BlockSpec(
            (bm, C), lambda f, g, gid, tid, go: (tid[g], 0)),
        scratch_shapes=[pltpu.VMEM((R, C), jnp.float32)],
    )
    y = pl.pallas_call(
        functools.partial(_gmm_body, bm=bm, n_ff=n_ff),
        grid_spec=grid_spec,
        out_shape=jax.ShapeDtypeStruct((R, C), jnp.float32),
    )(gid, tid, goff, x_sorted, W1, b1r, W2, b2r, sc2)
    return y


def kernel(x, gate_W, W1, b1, W2, b2):
    B, T, C = x.shape
    E, F = W1.shape[0], W1.shape[1]
    N = B * T
    R = 2 * N
    x_flat = x.reshape(N, C)

    e0, e1, w0, w1 = _gating(x_flat, gate_W)
    group_sizes, src_tokens, scales, pos = _routing(e0, e1, w0, w1, E)
    gid, tid, goff = _group_metadata(group_sizes, R, _BM, E)
    x_sorted = jnp.take(x_flat, src_tokens, axis=0)
    y = _gmm(x_sorted, W1, b1, W2, b2, scales, gid, tid, goff)
    out = jnp.take(y, pos[:N], axis=0) + jnp.take(y, pos[N:], axis=0)
    return out.reshape(B, T, C)


# speedup vs baseline: 2.0238x; 1.1286x over previous
import functools

import jax
import jax.numpy as jnp
from jax import lax
from jax.experimental import pallas as pl
from jax.experimental.pallas import tpu as pltpu
from jax.experimental.pallas import tpu_sc as plsc

_BM = 256
_BF = 512
_L = 16


def _gating_body(x_ref, gw_ref, e0_ref, e1_ref, w0_ref, w1_ref):
    xb = x_ref[...]
    logits = jax.lax.dot_general(
        xb, gw_ref[...], (((1,), (1,)), ((), ())),
        preferred_element_type=jnp.float32)
    m = jnp.max(logits, axis=-1, keepdims=True)
    p = jnp.exp(logits - m)
    w = p / jnp.sum(p, axis=-1, keepdims=True)
    E = w.shape[-1]
    iota = jax.lax.broadcasted_iota(jnp.int32, w.shape, 1)
    w0 = jnp.max(w, axis=-1, keepdims=True)
    i0 = jnp.min(jnp.where(w == w0, iota, E), axis=-1, keepdims=True)
    w_excl = jnp.where(iota == i0, -jnp.inf, w)
    w1v = jnp.max(w_excl, axis=-1, keepdims=True)
    i1 = jnp.min(jnp.where(w_excl == w1v, iota, E), axis=-1, keepdims=True)
    denom = w0 + w1v
    e0_ref[...] = i0
    e1_ref[...] = i1
    w0_ref[...] = w0 / denom
    w1_ref[...] = w1v / denom


def _gating(x_flat, gate_W):
    N, C = x_flat.shape
    E = gate_W.shape[0]
    outs = pl.pallas_call(
        _gating_body,
        grid=(1,),
        in_specs=[
            pl.BlockSpec((N, C), lambda i: (0, 0)),
            pl.BlockSpec((E, C), lambda i: (0, 0)),
        ],
        out_specs=[
            pl.BlockSpec((N, 1), lambda i: (0, 0)),
            pl.BlockSpec((N, 1), lambda i: (0, 0)),
            pl.BlockSpec((N, 1), lambda i: (0, 0)),
            pl.BlockSpec((N, 1), lambda i: (0, 0)),
        ],
        out_shape=[
            jax.ShapeDtypeStruct((N, 1), jnp.int32),
            jax.ShapeDtypeStruct((N, 1), jnp.int32),
            jax.ShapeDtypeStruct((N, 1), jnp.float32),
            jax.ShapeDtypeStruct((N, 1), jnp.float32),
        ],
    )(x_flat, gate_W)
    return [o.reshape(N) for o in outs]


def _routing_body(e0, e1, x_hbm, xs_hbm, pos_hbm, gs_hbm,
                  exp_v, pos_v, cnt_v, grid_sp, pos_sp, gridv, cur_v, tot_v,
                  pos_all, tok_v, xbuf, sem, *, N, S):
    core = lax.axis_index("c")
    w = lax.axis_index("s")
    n_chunk = S // _L
    lane = lax.iota(jnp.int32, _L)

    @pl.when(core == 0)
    def _run():
        half = w >= (N // S)
        off = w * S

        @pl.when(jnp.logical_not(half))
        def _():
            pltpu.sync_copy(e0.at[pl.ds(off, S)], exp_v)

        @pl.when(half)
        def _():
            pltpu.sync_copy(e1.at[pl.ds(off - N, S)], exp_v)

        def hist_step(c, counts):
            v = exp_v[pl.ds(c * _L, _L)]
            for e in range(8):
                n = jnp.sum((v == e).astype(jnp.int32))
                counts = jnp.where(lane == e, counts + n, counts)
            return counts

        cnt_v[...] = lax.fori_loop(0, n_chunk, hist_step,
                                   jnp.zeros((_L,), jnp.int32))
        pltpu.sync_copy(cnt_v, grid_sp.at[w])
        plsc.subcore_barrier()
        pltpu.sync_copy(grid_sp, gridv)

        totals = jnp.zeros((_L,), jnp.int32)
        prior = jnp.zeros((_L,), jnp.int32)
        for wp in range(16):
            row = gridv[wp]
            totals = totals + row
            sel = jnp.broadcast_to(wp < w, (_L,))
            prior = prior + jnp.where(sel, row, 0)
        start = plsc.cumsum(totals) - totals + prior
        cur_v[...] = start

        @pl.when(w == 0)
        def _():
            tot_v[...] = totals
            pltpu.sync_copy(tot_v, gs_hbm)

        def slot_step(c, _):
            v = exp_v[pl.ds(c * _L, _L)]
            base = plsc.load_gather(cur_v, [v])
            rank = jnp.zeros((_L,), jnp.int32)
            cur = cur_v[...]
            for e in range(8):
                m = v == e
                mi = m.astype(jnp.int32)
                r = plsc.cumsum(mi)
                rank = jnp.where(m, r - 1, rank)
                cur = jnp.where(lane == e, cur + jnp.sum(mi), cur)
            cur_v[...] = cur
            pos_v[c, :] = base + rank
            return 0

        lax.fori_loop(0, n_chunk, slot_step, 0)
        pltpu.sync_copy(pos_v, pos_hbm.at[w])

        pltpu.sync_copy(pos_v, pos_sp.at[w])
        for h in range(S // 64):
            for k in range(64 // _L):
                tok_v[h, pl.ds(k * _L, _L)] = jnp.zeros((_L,), jnp.int32)
        plsc.subcore_barrier()
        pltpu.sync_copy(pos_sp, pos_all)

        def inv_step(g, _):
            posc = pos_all[g // n_chunk, g % n_chunk, :]
            p = g * _L + lane
            tok = p - jnp.where(p >= N, N, 0)
            d = posc - w * S
            m = (d >= 0) & (d < S)
            dc = jnp.minimum(jnp.maximum(d, 0), S - 1)
            plsc.store_scatter(tok_v, [dc >> 6, dc & 63], tok, mask=m)
            return 0

        lax.fori_loop(0, (2 * N) // _L, inv_step, 0)

        for h in range(S // 64):
            pltpu.async_copy(x_hbm.at[tok_v.at[h]], xbuf, sem).wait()
            pltpu.sync_copy(xbuf, xs_hbm.at[pl.ds(w * S + h * 64, 64)])


def _routing(e0, e1, x_flat):
    N, C = x_flat.shape
    R = 2 * N
    S = R // 16
    n_chunk = S // _L
    mesh = plsc.VectorSubcoreMesh(core_axis_name="c", subcore_axis_name="s")
    xs, pos, gs = pl.kernel(
        functools.partial(_routing_body, N=N, S=S),
        out_type=[
            jax.ShapeDtypeStruct((R, C), jnp.float32),
            jax.ShapeDtypeStruct((16, n_chunk, _L), jnp.int32),
            jax.ShapeDtypeStruct((_L,), jnp.int32),
        ],
        mesh=mesh,
        scratch_types=[
            pltpu.VMEM((S,), jnp.int32),
            pltpu.VMEM((n_chunk, _L), jnp.int32),
            pltpu.VMEM((_L,), jnp.int32),
            pltpu.VMEM_SHARED((16, _L), jnp.int32),
            pltpu.VMEM_SHARED((16, n_chunk, _L), jnp.int32),
            pltpu.VMEM((16, _L), jnp.int32),
            pltpu.VMEM((_L,), jnp.int32),
            pltpu.VMEM((_L,), jnp.int32),
            pltpu.VMEM((16, n_chunk, _L), jnp.int32),
            pltpu.VMEM((S // 64, 64), jnp.int32),
            pltpu.VMEM((64, C), jnp.float32),
            pltpu.SemaphoreType.DMA,
        ],
        compiler_params=pltpu.CompilerParams(needs_layout_passes=False),
    )(e0, e1, x_flat)
    return xs, pos.reshape(R), gs


def _combine_body(y_hbm, pos_hbm, w0_hbm, w1_hbm, out_hbm,
                  p0v, p1v, w0v, w1v, rows0, rows1, sem0, sem1,
                  *, N, C, TW):
    core = lax.axis_index("c")
    sub = lax.axis_index("s")
    wid = sub * 2 + core
    base = wid * TW

    pltpu.sync_copy(pos_hbm.at[pl.ds(base, TW)], p0v)
    pltpu.sync_copy(pos_hbm.at[pl.ds(N + base, TW)], p1v)
    pltpu.sync_copy(w0_hbm.at[pl.ds(base, TW)], w0v)
    pltpu.sync_copy(w1_hbm.at[pl.ds(base, TW)], w1v)
    for k in range(TW // _L):
        sl = pl.ds(k * _L, _L)
        p0v[sl] = p0v[sl] & (2 * N - 1)
        p1v[sl] = p1v[sl] & (2 * N - 1)
    cp0 = pltpu.async_copy(y_hbm.at[p0v], rows0, sem0)
    cp1 = pltpu.async_copy(y_hbm.at[p1v], rows1, sem1)
    cp0.wait()
    cp1.wait()

    def body(r, carry):
        s0 = plsc.load_gather(w0v, [jnp.broadcast_to(r, (_L,))])
        s1 = plsc.load_gather(w1v, [jnp.broadcast_to(r, (_L,))])
        for c in range(C // _L):
            sl = pl.ds(c * _L, _L)
            rows0[r, sl] = s0 * rows0[r, sl] + s1 * rows1[r, sl]
        return carry

    lax.fori_loop(0, TW, body, 0)
    pltpu.sync_copy(rows0, out_hbm.at[pl.ds(base, TW)])


def _combine(y, pos, w0, w1):
    R, C = y.shape
    N = R // 2
    TW = N // 32
    mesh = plsc.VectorSubcoreMesh(core_axis_name="c", subcore_axis_name="s")
    out = pl.kernel(
        functools.partial(_combine_body, N=N, C=C, TW=TW),
        out_type=jax.ShapeDtypeStruct((N, C), jnp.float32),
        mesh=mesh,
        scratch_types=[
            pltpu.VMEM((TW,), jnp.int32),
            pltpu.VMEM((TW,), jnp.int32),
            pltpu.VMEM((TW,), jnp.float32),
            pltpu.VMEM((TW,), jnp.float32),
            pltpu.VMEM((TW, C), jnp.float32),
            pltpu.VMEM((TW, C), jnp.float32),
            pltpu.SemaphoreType.DMA,
            pltpu.SemaphoreType.DMA,
        ],
        compiler_params=pltpu.CompilerParams(needs_layout_passes=False),
    )(y, pos, w0, w1)
    return out


def _group_metadata(group_sizes, R, bm, E):
    g_max = R // bm + E - 1
    ends = jnp.cumsum(group_sizes)
    starts = ends - group_sizes
    ntiles = jnp.where(group_sizes > 0,
                       (ends - 1) // bm - starts // bm + 1, 0)
    total = jnp.sum(ntiles)
    tile_cum_excl = jnp.cumsum(ntiles) - ntiles
    posn = jnp.arange(g_max, dtype=jnp.int32)
    gid_raw = jnp.repeat(jnp.arange(E, dtype=jnp.int32), ntiles,
                         total_repeat_length=g_max)
    gid_last = jnp.take(gid_raw, total - 1)
    valid = posn < total
    gid = jnp.where(valid, gid_raw, gid_last)
    tid_raw = starts[gid] // bm + (posn - tile_cum_excl[gid])
    tid_last = jnp.take(tid_raw, total - 1)
    tid = jnp.where(valid, tid_raw, tid_last).astype(jnp.int32)
    goff = jnp.concatenate([jnp.zeros(1, jnp.int32),
                            ends.astype(jnp.int32)])
    return gid.astype(jnp.int32), tid, goff


def _gmm_body(gid_ref, tid_ref, goff_ref,
              xs_ref, w1_ref, b1_ref, w2_ref, b2_ref,
              y_ref, acc_ref, *, bm, n_ff):
    f = pl.program_id(0)
    g = pl.program_id(1)
    e = gid_ref[g]
    t = tid_ref[g]

    xb = xs_ref[...]
    w1b = w1_ref[0]
    h = jax.lax.dot_general(xb, w1b, (((1,), (1,)), ((), ())),
                            preferred_element_type=jnp.float32)
    h = jnp.maximum(h + b1_ref[0], 0.0)
    w2b = w2_ref[0]
    part = jax.lax.dot_general(h, w2b, (((1,), (1,)), ((), ())),
                               preferred_element_type=jnp.float32)

    rows = t * bm + jax.lax.broadcasted_iota(jnp.int32, (bm, 1), 0)
    mask = (rows >= goff_ref[e]) & (rows < goff_ref[e + 1])

    sl = pl.ds(t * bm, bm)
    prev = acc_ref[sl, :]
    acc = jnp.where(mask, jnp.where(f == 0, part, prev + part), prev)
    acc_ref[sl, :] = acc

    @pl.when(f == n_ff - 1)
    def _store():
        y_ref[...] = jnp.where(mask, acc + b2_ref[0], y_ref[...])


def _gmm(x_sorted, W1, b1, W2, b2, gid, tid, goff):
    R, C = x_sorted.shape
    E, F = W1.shape[0], W1.shape[1]
    bm, bf = _BM, _BF
    n_ff = F // bf
    g_max = gid.shape[0]
    b1r = b1.reshape(E * n_ff, 1, bf)
    b2r = b2.reshape(E, 1, C)

    grid_spec = pltpu.PrefetchScalarGridSpec(
        num_scalar_prefetch=3,
        grid=(n_ff, g_max),
        in_specs=[
            pl.BlockSpec((bm, C), lambda f, g, gid, tid, go: (tid[g], 0)),
            pl.BlockSpec((1, bf, C), lambda f, g, gid, tid, go: (gid[g], f, 0)),
            pl.BlockSpec((1, 1, bf),
                         lambda f, g, gid, tid, go: (gid[g] * n_ff + f, 0, 0)),
            pl.BlockSpec((1, C, bf), lambda f, g, gid, tid, go: (gid[g], 0, f)),
            pl.BlockSpec((1, 1, C), lambda f, g, gid, tid, go: (gid[g], 0, 0)),
        ],
        out_specs=pl.BlockSpec(
            (bm, C), lambda f, g, gid, tid, go: (tid[g], 0)),
        scratch_shapes=[pltpu.VMEM((R, C), jnp.float32)],
    )
    y = pl.pallas_call(
        functools.partial(_gmm_body, bm=bm, n_ff=n_ff),
        grid_spec=grid_spec,
        out_shape=jax.ShapeDtypeStruct((R, C), jnp.float32),
    )(gid, tid, goff, x_sorted, W1, b1r, W2, b2r)
    return y


def kernel(x, gate_W, W1, b1, W2, b2):
    B, T, C = x.shape
    E, F = W1.shape[0], W1.shape[1]
    N = B * T
    R = 2 * N
    x_flat = x.reshape(N, C)

    e0, e1, w0, w1 = _gating(x_flat, gate_W)
    x_sorted, pos, gs16 = _routing(e0, e1, x_flat)
    gid, tid, goff = _group_metadata(gs16[:E], R, _BM, E)
    y = _gmm(x_sorted, W1, b1, W2, b2, gid, tid, goff)
    out = _combine(y, pos, w0, w1)
    return out.reshape(B, T, C)


# speedup vs baseline: 3.0934x; 1.5285x over previous
import functools

import jax
import jax.numpy as jnp
from jax import lax
from jax.experimental import pallas as pl
from jax.experimental.pallas import tpu as pltpu
from jax.experimental.pallas import tpu_sc as plsc

_BM = 256
_BF = 512
_L = 16


def _gating_body(x_ref, gw_ref, e0_ref, e1_ref, w0_ref, w1_ref):
    xb = x_ref[...]
    logits = jax.lax.dot_general(
        xb, gw_ref[...], (((1,), (1,)), ((), ())),
        preferred_element_type=jnp.float32)
    m = jnp.max(logits, axis=-1, keepdims=True)
    p = jnp.exp(logits - m)
    w = p / jnp.sum(p, axis=-1, keepdims=True)
    E = w.shape[-1]
    iota = jax.lax.broadcasted_iota(jnp.int32, w.shape, 1)
    w0 = jnp.max(w, axis=-1, keepdims=True)
    i0 = jnp.min(jnp.where(w == w0, iota, E), axis=-1, keepdims=True)
    w_excl = jnp.where(iota == i0, -jnp.inf, w)
    w1v = jnp.max(w_excl, axis=-1, keepdims=True)
    i1 = jnp.min(jnp.where(w_excl == w1v, iota, E), axis=-1, keepdims=True)
    denom = w0 + w1v
    e0_ref[...] = i0
    e1_ref[...] = i1
    w0_ref[...] = w0 / denom
    w1_ref[...] = w1v / denom


def _gating(x_flat, gate_W):
    N, C = x_flat.shape
    E = gate_W.shape[0]
    outs = pl.pallas_call(
        _gating_body,
        grid=(1,),
        in_specs=[
            pl.BlockSpec((N, C), lambda i: (0, 0)),
            pl.BlockSpec((E, C), lambda i: (0, 0)),
        ],
        out_specs=[
            pl.BlockSpec((N, 1), lambda i: (0, 0)),
            pl.BlockSpec((N, 1), lambda i: (0, 0)),
            pl.BlockSpec((N, 1), lambda i: (0, 0)),
            pl.BlockSpec((N, 1), lambda i: (0, 0)),
        ],
        out_shape=[
            jax.ShapeDtypeStruct((N, 1), jnp.int32),
            jax.ShapeDtypeStruct((N, 1), jnp.int32),
            jax.ShapeDtypeStruct((N, 1), jnp.float32),
            jax.ShapeDtypeStruct((N, 1), jnp.float32),
        ],
    )(x_flat, gate_W)
    return [o.reshape(N) for o in outs]


def _routing_body(e0, e1, x_hbm, xs_hbm, pos_hbm, gs_hbm,
                  exp_v, pos_v, cnt_v, grid_sp, pos_sp, gridv, cur_v, tot_v,
                  pos_all, tok_v, xbuf, sem, *, N, S):
    core = lax.axis_index("c")
    w = lax.axis_index("s")
    n_chunk = S // _L
    lane = lax.iota(jnp.int32, _L)

    @pl.when(core == 0)
    def _run():
        half = w >= (N // S)
        off = w * S

        @pl.when(jnp.logical_not(half))
        def _():
            pltpu.sync_copy(e0.at[pl.ds(off, S)], exp_v)

        @pl.when(half)
        def _():
            pltpu.sync_copy(e1.at[pl.ds(off - N, S)], exp_v)

        def hist_step(c, counts):
            v = exp_v[pl.ds(c * _L, _L)]
            for e in range(8):
                n = jnp.sum((v == e).astype(jnp.int32))
                counts = jnp.where(lane == e, counts + n, counts)
            return counts

        cnt_v[...] = lax.fori_loop(0, n_chunk, hist_step,
                                   jnp.zeros((_L,), jnp.int32))
        pltpu.sync_copy(cnt_v, grid_sp.at[w])
        plsc.subcore_barrier()
        pltpu.sync_copy(grid_sp, gridv)

        totals = jnp.zeros((_L,), jnp.int32)
        prior = jnp.zeros((_L,), jnp.int32)
        for wp in range(16):
            row = gridv[wp]
            totals = totals + row
            sel = jnp.broadcast_to(wp < w, (_L,))
            prior = prior + jnp.where(sel, row, 0)
        start = plsc.cumsum(totals) - totals + prior
        cur_v[...] = start

        @pl.when(w == 0)
        def _():
            tot_v[...] = totals
            pltpu.sync_copy(tot_v, gs_hbm)

        def slot_step(c, _):
            v = exp_v[pl.ds(c * _L, _L)]
            base = plsc.load_gather(cur_v, [v])
            rank = jnp.zeros((_L,), jnp.int32)
            cur = cur_v[...]
            for e in range(8):
                m = v == e
                mi = m.astype(jnp.int32)
                r = plsc.cumsum(mi)
                rank = jnp.where(m, r - 1, rank)
                cur = jnp.where(lane == e, cur + jnp.sum(mi), cur)
            cur_v[...] = cur
            pos_v[c, :] = base + rank
            return 0

        lax.fori_loop(0, n_chunk, slot_step, 0)
        pltpu.sync_copy(pos_v, pos_hbm.at[w])

        pltpu.sync_copy(pos_v, pos_sp.at[w])
        for h in range(S // 64):
            for k in range(64 // _L):
                tok_v[h, pl.ds(k * _L, _L)] = jnp.zeros((_L,), jnp.int32)
        plsc.subcore_barrier()
        pltpu.sync_copy(pos_sp, pos_all)

        def inv_step(g, _):
            posc = pos_all[g // n_chunk, g % n_chunk, :]
            p = g * _L + lane
            tok = p - jnp.where(p >= N, N, 0)
            d = posc - w * S
            m = (d >= 0) & (d < S)
            dc = jnp.minimum(jnp.maximum(d, 0), S - 1)
            plsc.store_scatter(tok_v, [dc >> 6, dc & 63], tok, mask=m)
            return 0

        lax.fori_loop(0, (2 * N) // _L, inv_step, 0)

        for h in range(S // 64):
            pltpu.async_copy(x_hbm.at[tok_v.at[h]], xbuf, sem).wait()
            pltpu.sync_copy(xbuf, xs_hbm.at[pl.ds(w * S + h * 64, 64)])


def _routing(e0, e1, x_flat):
    N, C = x_flat.shape
    R = 2 * N
    S = R // 16
    n_chunk = S // _L
    mesh = plsc.VectorSubcoreMesh(core_axis_name="c", subcore_axis_name="s")
    xs, pos, gs = pl.kernel(
        functools.partial(_routing_body, N=N, S=S),
        out_type=[
            jax.ShapeDtypeStruct((R, C), jnp.float32),
            jax.ShapeDtypeStruct((16, n_chunk, _L), jnp.int32),
            jax.ShapeDtypeStruct((_L,), jnp.int32),
        ],
        mesh=mesh,
        scratch_types=[
            pltpu.VMEM((S,), jnp.int32),
            pltpu.VMEM((n_chunk, _L), jnp.int32),
            pltpu.VMEM((_L,), jnp.int32),
            pltpu.VMEM_SHARED((16, _L), jnp.int32),
            pltpu.VMEM_SHARED((16, n_chunk, _L), jnp.int32),
            pltpu.VMEM((16, _L), jnp.int32),
            pltpu.VMEM((_L,), jnp.int32),
            pltpu.VMEM((_L,), jnp.int32),
            pltpu.VMEM((16, n_chunk, _L), jnp.int32),
            pltpu.VMEM((S // 64, 64), jnp.int32),
            pltpu.VMEM((64, C), jnp.float32),
            pltpu.SemaphoreType.DMA,
        ],
        compiler_params=pltpu.CompilerParams(needs_layout_passes=False),
    )(e0, e1, x_flat)
    return xs, pos.reshape(R), gs


def _combine_body(y_hbm, pos_hbm, w0_hbm, w1_hbm, out_hbm,
                  p0v, p1v, w0v, w1v, rows0, rows1, sem0, sem1,
                  *, N, C, TW):
    core = lax.axis_index("c")
    sub = lax.axis_index("s")
    wid = sub * 2 + core
    base = wid * TW

    pltpu.sync_copy(pos_hbm.at[pl.ds(base, TW)], p0v)
    pltpu.sync_copy(pos_hbm.at[pl.ds(N + base, TW)], p1v)
    pltpu.sync_copy(w0_hbm.at[pl.ds(base, TW)], w0v)
    pltpu.sync_copy(w1_hbm.at[pl.ds(base, TW)], w1v)
    for k in range(TW // _L):
        sl = pl.ds(k * _L, _L)
        p0v[sl] = p0v[sl] & (2 * N - 1)
        p1v[sl] = p1v[sl] & (2 * N - 1)
    cp0 = pltpu.async_copy(y_hbm.at[p0v], rows0, sem0)
    cp1 = pltpu.async_copy(y_hbm.at[p1v], rows1, sem1)
    cp0.wait()
    cp1.wait()

    def body(r, carry):
        s0 = plsc.load_gather(w0v, [jnp.broadcast_to(r, (_L,))])
        s1 = plsc.load_gather(w1v, [jnp.broadcast_to(r, (_L,))])
        for c in range(C // _L):
            sl = pl.ds(c * _L, _L)
            rows0[r, sl] = s0 * rows0[r, sl] + s1 * rows1[r, sl]
        return carry

    lax.fori_loop(0, TW, body, 0)
    pltpu.sync_copy(rows0, out_hbm.at[pl.ds(base, TW)])


def _combine(y, pos, w0, w1):
    R, C = y.shape
    N = R // 2
    TW = N // 32
    mesh = plsc.VectorSubcoreMesh(core_axis_name="c", subcore_axis_name="s")
    out = pl.kernel(
        functools.partial(_combine_body, N=N, C=C, TW=TW),
        out_type=jax.ShapeDtypeStruct((N, C), jnp.float32),
        mesh=mesh,
        scratch_types=[
            pltpu.VMEM((TW,), jnp.int32),
            pltpu.VMEM((TW,), jnp.int32),
            pltpu.VMEM((TW,), jnp.float32),
            pltpu.VMEM((TW,), jnp.float32),
            pltpu.VMEM((TW, C), jnp.float32),
            pltpu.VMEM((TW, C), jnp.float32),
            pltpu.SemaphoreType.DMA,
            pltpu.SemaphoreType.DMA,
        ],
        compiler_params=pltpu.CompilerParams(needs_layout_passes=False),
    )(y, pos, w0, w1)
    return out


def _group_metadata(group_sizes, R, bm, E):
    g_max = R // bm + E - 1
    ends = jnp.cumsum(group_sizes)
    starts = ends - group_sizes
    ntiles = jnp.where(group_sizes > 0,
                       (ends - 1) // bm - starts // bm + 1, 0)
    total = jnp.sum(ntiles)
    tile_cum_excl = jnp.cumsum(ntiles) - ntiles
    posn = jnp.arange(g_max, dtype=jnp.int32)
    gid_raw = jnp.repeat(jnp.arange(E, dtype=jnp.int32), ntiles,
                         total_repeat_length=g_max)
    gid_last = jnp.take(gid_raw, total - 1)
    valid = posn < total
    gid = jnp.where(valid, gid_raw, gid_last)
    tid_raw = starts[gid] // bm + (posn - tile_cum_excl[gid])
    tid_last = jnp.take(tid_raw, total - 1)
    tid = jnp.where(valid, tid_raw, tid_last).astype(jnp.int32)
    goff = jnp.concatenate([jnp.zeros(1, jnp.int32),
                            ends.astype(jnp.int32)])
    return gid.astype(jnp.int32), tid, goff


def _gmm_body(gid_ref, tid_ref, goff_ref,
              xs_ref, w1_ref, b1_ref, w2_ref, b2_ref,
              y_ref, *, bm):
    g = pl.program_id(0)
    e = gid_ref[g]
    t = tid_ref[g]

    xb = xs_ref[...].astype(jnp.bfloat16)
    w1b = w1_ref[0].astype(jnp.bfloat16)
    h = jax.lax.dot_general(xb, w1b, (((1,), (1,)), ((), ())),
                            preferred_element_type=jnp.float32)
    h = jnp.maximum(h + b1_ref[0], 0.0)
    w2b = w2_ref[0].astype(jnp.bfloat16)
    part = jax.lax.dot_general(h.astype(jnp.bfloat16), w2b,
                               (((1,), (1,)), ((), ())),
                               preferred_element_type=jnp.float32)

    rows = t * bm + jax.lax.broadcasted_iota(jnp.int32, (bm, 1), 0)
    mask = (rows >= goff_ref[e]) & (rows < goff_ref[e + 1])
    y_ref[...] = jnp.where(mask, part + b2_ref[0], y_ref[...])


def _gmm(x_sorted, W1, b1, W2, b2, gid, tid, goff):
    R, C = x_sorted.shape
    E, F = W1.shape[0], W1.shape[1]
    bm = _BM
    g_max = gid.shape[0]
    b1r = b1.reshape(E, 1, F)
    b2r = b2.reshape(E, 1, C)

    grid_spec = pltpu.PrefetchScalarGridSpec(
        num_scalar_prefetch=3,
        grid=(g_max,),
        in_specs=[
            pl.BlockSpec((bm, C), lambda g, gid, tid, go: (tid[g], 0)),
            pl.BlockSpec((1, F, C), lambda g, gid, tid, go: (gid[g], 0, 0)),
            pl.BlockSpec((1, 1, F), lambda g, gid, tid, go: (gid[g], 0, 0)),
            pl.BlockSpec((1, C, F), lambda g, gid, tid, go: (gid[g], 0, 0)),
            pl.BlockSpec((1, 1, C), lambda g, gid, tid, go: (gid[g], 0, 0)),
        ],
        out_specs=pl.BlockSpec(
            (bm, C), lambda g, gid, tid, go: (tid[g], 0)),
    )
    y = pl.pallas_call(
        functools.partial(_gmm_body, bm=bm),
        grid_spec=grid_spec,
        out_shape=jax.ShapeDtypeStruct((R, C), jnp.float32),
    )(gid, tid, goff, x_sorted, W1, b1r, W2, b2r)
    return y


def kernel(x, gate_W, W1, b1, W2, b2):
    B, T, C = x.shape
    E, F = W1.shape[0], W1.shape[1]
    N = B * T
    R = 2 * N
    x_flat = x.reshape(N, C)

    e0, e1, w0, w1 = _gating(x_flat, gate_W)
    x_sorted, pos, gs16 = _routing(e0, e1, x_flat)
    gid, tid, goff = _group_metadata(gs16[:E], R, _BM, E)
    y = _gmm(x_sorted, W1, b1, W2, b2, gid, tid, goff)
    out = _combine(y, pos, w0, w1)
    return out.reshape(B, T, C)
